# baseline (speedup 1.0000x reference)
import functools

import jax
import jax.numpy as jnp
from jax.experimental import pallas as pl
from jax.experimental.pallas import tpu as pltpu


def _mm_kernel(x_ref, w_ref, b_ref, o_ref):
    o_ref[...] = jnp.dot(x_ref[...], w_ref[...],
                         preferred_element_type=jnp.float32) + b_ref[...]


def _matmul_bias(x, W, b, block=1000):
    n, d = x.shape
    grid = (n // block,)
    return pl.pallas_call(
        _mm_kernel,
        grid=grid,
        in_specs=[
            pl.BlockSpec((block, d), lambda i: (i, 0)),
            pl.BlockSpec((d, d), lambda i: (0, 0)),
            pl.BlockSpec((1, d), lambda i: (0, 0)),
        ],
        out_specs=pl.BlockSpec((block, d), lambda i: (i, 0)),
        out_shape=jax.ShapeDtypeStruct((n, d), jnp.float32),
    )(x, W, b.reshape(1, d))


def _mean_agg(Wh, src, dst, ew, n_dst):
    m = Wh[src] * ew[:, None]
    s = jax.ops.segment_sum(m, dst, num_segments=n_dst)
    cnt = jax.ops.segment_sum(jnp.ones_like(ew), dst, num_segments=n_dst)
    return jnp.where(cnt[:, None] > 0, s / jnp.maximum(cnt, 1.0)[:, None], 0.0)


def kernel(x_word, x_topic, effect, src_ww, dst_ww, ew_ww, src_wt, dst_wt, ew_wt, src_wd, dst_wd, ew_wd, src_td, dst_td, ew_td, src_tt, dst_tt, ew_tt, W_ww, b_ww, W_wt, b_wt, W_wd, b_wd, W_td, b_td, W_tt, b_tt, W_causal, W_noise):
    N_WORD, N_TOPIC, N_DOC = 50000, 10000, 10000
    pos = (effect > 0).astype(jnp.float32)[:, None]
    neg = (effect < 0).astype(jnp.float32)[:, None]
    Wh_ww = _matmul_bias(x_word, W_ww, b_ww)
    Wh_wt = _matmul_bias(x_word, W_wt, b_wt)
    Wh_wd = _matmul_bias(x_word, W_wd, b_wd)
    extra = _matmul_bias(x_topic * pos, W_causal, jnp.zeros_like(b_ww)) - \
        _matmul_bias(x_topic * neg, W_noise, jnp.zeros_like(b_ww))
    Wh_td = _matmul_bias(x_topic, W_td, b_td) + extra
    Wh_tt = _matmul_bias(x_topic, W_tt, b_tt) + extra
    h_word = _mean_agg(Wh_ww, src_ww, dst_ww, ew_ww, N_WORD)
    h_topic = _mean_agg(Wh_wt, src_wt, dst_wt, ew_wt, N_TOPIC) + \
        _mean_agg(Wh_tt, src_tt, dst_tt, ew_tt, N_TOPIC)
    h_doc = _mean_agg(Wh_wd, src_wd, dst_wd, ew_wd, N_DOC) + \
        _mean_agg(Wh_td, src_td, dst_td, ew_td, N_DOC)
    return (h_word, h_topic, h_doc)
